# baseline (speedup 1.0000x reference)
import functools

import jax
import jax.numpy as jnp
from jax.experimental import pallas as pl
from jax.experimental.pallas import tpu as pltpu

TOKEN_BLOCK = 2048


def _gating_block(x_ref, w1t_ref, b1_ref, w2t_ref, b2_ref, out_ref):
    xb = x_ref[...].astype(jnp.bfloat16)
    h = jnp.tanh(
        jnp.dot(xb, w1t_ref[...], preferred_element_type=jnp.float32)
        + b1_ref[...]
    )
    out_ref[...] = (
        jnp.dot(h.astype(jnp.bfloat16), w2t_ref[...],
                preferred_element_type=jnp.float32)
        + b2_ref[...]
    )


@jax.jit
def _gating(x, w1t, b1, w2t, b2):
    tokens = x.shape[0]
    num_experts = w1t.shape[1]
    grid = (tokens // TOKEN_BLOCK,)
    gates = pl.pallas_call(
        _gating_block,
        grid=grid,
        in_specs=[
            pl.BlockSpec((TOKEN_BLOCK, x.shape[1]), lambda i: (i, 0)),
            pl.BlockSpec((x.shape[1], num_experts), lambda i: (0, 0)),
            pl.BlockSpec((1, num_experts), lambda i: (0, 0)),
            pl.BlockSpec((num_experts, num_experts), lambda i: (0, 0)),
            pl.BlockSpec((1, num_experts), lambda i: (0, 0)),
        ],
        out_specs=pl.BlockSpec((TOKEN_BLOCK, num_experts), lambda i: (i, 0)),
        out_shape=jax.ShapeDtypeStruct((tokens, num_experts), jnp.float32),
        compiler_params=pltpu.CompilerParams(
            dimension_semantics=("parallel",),
        ),
    )(x, w1t, b1, w2t, b2)
    return gates


def kernel(x, W1, b1, W2, b2, train):
    w1t = W1.T.astype(jnp.bfloat16)
    w2t = W2.T.astype(jnp.bfloat16)
    gates = _gating(x, w1t, b1.reshape(1, -1), w2t, b2.reshape(1, -1))
    return (gates, gates)


# speedup vs baseline: 1.0474x; 1.0474x over previous
import functools

import jax
import jax.numpy as jnp
from jax.experimental import pallas as pl
from jax.experimental.pallas import tpu as pltpu

TOKEN_BLOCK = 4096


def _gating_block(x_ref, w1t_ref, b1_ref, w2t_ref, b2_ref, out_ref):
    xb = x_ref[...].astype(jnp.bfloat16)
    h = jnp.tanh(
        jnp.dot(xb, w1t_ref[...], preferred_element_type=jnp.float32)
        + b1_ref[...]
    )
    out_ref[...] = (
        jnp.dot(h.astype(jnp.bfloat16), w2t_ref[...],
                preferred_element_type=jnp.float32)
        + b2_ref[...]
    )


@jax.jit
def _gating(x, w1t, b1, w2t, b2):
    tokens = x.shape[0]
    num_experts = w1t.shape[1]
    grid = (tokens // TOKEN_BLOCK,)
    gates = pl.pallas_call(
        _gating_block,
        grid=grid,
        in_specs=[
            pl.BlockSpec((TOKEN_BLOCK, x.shape[1]), lambda i: (i, 0)),
            pl.BlockSpec((x.shape[1], num_experts), lambda i: (0, 0)),
            pl.BlockSpec((1, num_experts), lambda i: (0, 0)),
            pl.BlockSpec((num_experts, num_experts), lambda i: (0, 0)),
            pl.BlockSpec((1, num_experts), lambda i: (0, 0)),
        ],
        out_specs=pl.BlockSpec((TOKEN_BLOCK, num_experts), lambda i: (i, 0)),
        out_shape=jax.ShapeDtypeStruct((tokens, num_experts), jnp.float32),
        compiler_params=pltpu.CompilerParams(
            dimension_semantics=("parallel",),
        ),
    )(x, w1t, b1, w2t, b2)
    return gates


def kernel(x, W1, b1, W2, b2, train):
    w1t = W1.T.astype(jnp.bfloat16)
    w2t = W2.T.astype(jnp.bfloat16)
    gates = _gating(x, w1t, b1.reshape(1, -1), w2t, b2.reshape(1, -1))
    return (gates, gates)
